# initial kernel scaffold (unmeasured)
import jax
import jax.numpy as jnp
from jax import lax
from jax.experimental import pallas as pl
from jax.experimental.pallas import tpu as pltpu

N_DEV = 16


def kernel(x, W1, W2):
    m, k = x.shape
    _, h_per = W1.shape
    _, n = W2.shape
    chunk = m // N_DEV
    nhops = N_DEV - 1

    def body(x_ref, w1_ref, w2_ref, out_ref, h_ref, acc_ref, rs_buf,
             rs_sems, ag_sems):
        my = lax.axis_index("i")
        left = lax.rem(my + N_DEV - 1, N_DEV)
        right = lax.rem(my + 1, N_DEV)

        barrier_sem = pltpu.get_barrier_semaphore()
        for nbr in (left, right):
            pl.semaphore_signal(
                barrier_sem, inc=1,
                device_id=(nbr,), device_id_type=pl.DeviceIdType.MESH,
            )
        pl.semaphore_wait(barrier_sem, 2)

        h_ref[...] = jnp.maximum(
            jnp.dot(x_ref[...], w1_ref[...],
                    preferred_element_type=jnp.float32),
            0.0,
        )
        out_ref[...] = jnp.dot(
            h_ref[...], w2_ref[...], preferred_element_type=jnp.float32,
        ).reshape(N_DEV, chunk, n)

        acc_ref[...] = out_ref[my]
        for s in range(nhops):
            rdma = pltpu.make_async_remote_copy(
                src_ref=acc_ref,
                dst_ref=rs_buf.at[s],
                send_sem=rs_sems.at[0, s],
                recv_sem=rs_sems.at[1, s],
                device_id=(right,),
                device_id_type=pl.DeviceIdType.MESH,
            )
            rdma.start()
            rdma.wait()
            idx = lax.rem(my + N_DEV - s - 1, N_DEV)
            acc_ref[...] = rs_buf[s] + out_ref[idx]

        out_ref[right] = acc_ref[...]

        for s in range(nhops):
            src_idx = lax.rem(my + 1 - s + N_DEV, N_DEV)
            rdma = pltpu.make_async_remote_copy(
                src_ref=out_ref.at[src_idx],
                dst_ref=out_ref.at[src_idx],
                send_sem=ag_sems.at[0, s],
                recv_sem=ag_sems.at[1, s],
                device_id=(right,),
                device_id_type=pl.DeviceIdType.MESH,
            )
            rdma.start()
            rdma.wait()

    out = pl.pallas_call(
        body,
        out_shape=jax.ShapeDtypeStruct((N_DEV, chunk, n), jnp.float32),
        in_specs=[
            pl.BlockSpec(memory_space=pltpu.VMEM),
            pl.BlockSpec(memory_space=pltpu.VMEM),
            pl.BlockSpec(memory_space=pltpu.VMEM),
        ],
        out_specs=pl.BlockSpec(memory_space=pltpu.VMEM),
        scratch_shapes=[
            pltpu.VMEM((m, h_per), jnp.float32),
            pltpu.VMEM((chunk, n), jnp.float32),
            pltpu.VMEM((nhops, chunk, n), jnp.float32),
            pltpu.SemaphoreType.DMA((2, nhops)),
            pltpu.SemaphoreType.DMA((2, nhops)),
        ],
        compiler_params=pltpu.CompilerParams(collective_id=0),
    )(x, W1, W2)
    return out.reshape(m, n)


# baseline (device time: 114640 ns/iter reference)
import jax
import jax.numpy as jnp
from jax import lax
from jax.experimental import pallas as pl
from jax.experimental.pallas import tpu as pltpu

N_DEV = 16


def kernel(x, W1, W2):
    m, k = x.shape
    _, h_per = W1.shape
    _, n = W2.shape
    chunk = m // N_DEV
    nhops = N_DEV - 1

    def body(x_ref, w1_ref, w2_ref, out_ref, acc_ref, rs_buf, rs_sems,
             ag_sems):
        my = lax.axis_index("i")
        left = lax.rem(my + N_DEV - 1, N_DEV)
        right = lax.rem(my + 1, N_DEV)

        barrier_sem = pltpu.get_barrier_semaphore()
        for nbr in (left, right):
            pl.semaphore_signal(
                barrier_sem, inc=1,
                device_id=(nbr,), device_id_type=pl.DeviceIdType.MESH,
            )
        pl.semaphore_wait(barrier_sem, 2)

        def compute_chunk(idx):
            xa = x_ref[pl.ds(idx * chunk, chunk), :]
            hh = jnp.maximum(
                jnp.dot(xa, w1_ref[...], preferred_element_type=jnp.float32),
                0.0,
            )
            return jnp.dot(hh, w2_ref[...],
                           preferred_element_type=jnp.float32)

        acc_ref[0] = compute_chunk(my)
        for s in range(nhops):
            cur = s % 2
            rdma = pltpu.make_async_remote_copy(
                src_ref=acc_ref.at[cur],
                dst_ref=rs_buf.at[s],
                send_sem=rs_sems.at[0, s],
                recv_sem=rs_sems.at[1, s],
                device_id=(right,),
                device_id_type=pl.DeviceIdType.MESH,
            )
            rdma.start()
            p = compute_chunk(lax.rem(my + N_DEV - s - 1, N_DEV))
            rdma.wait()
            acc_ref[1 - cur] = rs_buf[s] + p

        out_ref[right] = acc_ref[1]

        for s in range(nhops):
            src_idx = lax.rem(my + 1 - s + N_DEV, N_DEV)
            rdma = pltpu.make_async_remote_copy(
                src_ref=out_ref.at[src_idx],
                dst_ref=out_ref.at[src_idx],
                send_sem=ag_sems.at[0, s],
                recv_sem=ag_sems.at[1, s],
                device_id=(right,),
                device_id_type=pl.DeviceIdType.MESH,
            )
            rdma.start()
            rdma.wait()

    out = pl.pallas_call(
        body,
        out_shape=jax.ShapeDtypeStruct((N_DEV, chunk, n), jnp.float32),
        in_specs=[
            pl.BlockSpec(memory_space=pltpu.VMEM),
            pl.BlockSpec(memory_space=pltpu.VMEM),
            pl.BlockSpec(memory_space=pltpu.VMEM),
        ],
        out_specs=pl.BlockSpec(memory_space=pltpu.VMEM),
        scratch_shapes=[
            pltpu.VMEM((2, chunk, n), jnp.float32),
            pltpu.VMEM((nhops, chunk, n), jnp.float32),
            pltpu.SemaphoreType.DMA((2, nhops)),
            pltpu.SemaphoreType.DMA((2, nhops)),
        ],
        compiler_params=pltpu.CompilerParams(collective_id=0),
    )(x, W1, W2)
    return out.reshape(m, n)


# device time: 67013 ns/iter; 1.7107x vs baseline; 1.7107x over previous
import jax
import jax.numpy as jnp
from jax import lax
from jax.experimental import pallas as pl
from jax.experimental.pallas import tpu as pltpu

N_DEV = 16


def kernel(x, W1, W2):
    m, k = x.shape
    _, h_per = W1.shape
    _, n = W2.shape
    chunk = m // N_DEV
    npeer = N_DEV - 1

    def body(x_ref, w1_ref, w2_ref, out_ref, p_ref, h_ref, rbuf,
             p1_send, p1_recv, p2_send, p2_recv):
        my = lax.axis_index("i")

        barrier_sem = pltpu.get_barrier_semaphore()
        for j in range(1, N_DEV):
            pl.semaphore_signal(
                barrier_sem, inc=1,
                device_id=(lax.rem(my + j, N_DEV),),
                device_id_type=pl.DeviceIdType.MESH,
            )
        pl.semaphore_wait(barrier_sem, npeer)

        h_ref[...] = jnp.maximum(
            jnp.dot(x_ref[...], w1_ref[...],
                    preferred_element_type=jnp.float32),
            0.0,
        )
        p_ref[...] = jnp.dot(
            h_ref[...], w2_ref[...], preferred_element_type=jnp.float32,
        ).reshape(N_DEV, chunk, n)

        p1 = []
        for j in range(1, N_DEV):
            t = lax.rem(my + j, N_DEV)
            rdma = pltpu.make_async_remote_copy(
                src_ref=p_ref.at[t],
                dst_ref=rbuf.at[j - 1],
                send_sem=p1_send.at[j - 1],
                recv_sem=p1_recv.at[j - 1],
                device_id=(t,),
                device_id_type=pl.DeviceIdType.MESH,
            )
            rdma.start()
            p1.append(rdma)

        acc = p_ref[my]
        for j in range(1, N_DEV):
            p1[j - 1].wait_recv()
            acc = acc + rbuf[j - 1]
        out_ref[my] = acc

        p2 = []
        for j in range(1, N_DEV):
            t = lax.rem(my + j, N_DEV)
            rdma = pltpu.make_async_remote_copy(
                src_ref=out_ref.at[my],
                dst_ref=out_ref.at[my],
                send_sem=p2_send.at[j - 1],
                recv_sem=p2_recv.at[j - 1],
                device_id=(t,),
                device_id_type=pl.DeviceIdType.MESH,
            )
            rdma.start()
            p2.append(rdma)

        for j in range(1, N_DEV):
            p2[j - 1].wait_recv()
        for j in range(1, N_DEV):
            p1[j - 1].wait_send()
            p2[j - 1].wait_send()

    out = pl.pallas_call(
        body,
        out_shape=jax.ShapeDtypeStruct((N_DEV, chunk, n), jnp.float32),
        in_specs=[
            pl.BlockSpec(memory_space=pltpu.VMEM),
            pl.BlockSpec(memory_space=pltpu.VMEM),
            pl.BlockSpec(memory_space=pltpu.VMEM),
        ],
        out_specs=pl.BlockSpec(memory_space=pltpu.VMEM),
        scratch_shapes=[
            pltpu.VMEM((N_DEV, chunk, n), jnp.float32),
            pltpu.VMEM((m, h_per), jnp.float32),
            pltpu.VMEM((npeer, chunk, n), jnp.float32),
            pltpu.SemaphoreType.DMA((npeer,)),
            pltpu.SemaphoreType.DMA((npeer,)),
            pltpu.SemaphoreType.DMA((npeer,)),
            pltpu.SemaphoreType.DMA((npeer,)),
        ],
        compiler_params=pltpu.CompilerParams(collective_id=0),
    )(x, W1, W2)
    return out.reshape(m, n)
